# Initial kernel scaffold; baseline (speedup 1.0000x reference)
import jax
import jax.numpy as jnp
from jax.experimental import pallas as pl


def kernel(z, embeds):
    raise NotImplementedError("write your pallas kernel here")


# speedup vs baseline: 1.7257x; 1.7257x over previous
import functools

import jax
import jax.numpy as jnp
from jax import lax
from jax.experimental import pallas as pl
from jax.experimental.pallas import tpu as pltpu
from jax.experimental.pallas import tpu_sc as plsc

D = 32
K = 8192
NCB = 4
TB = 256

NC = 2
NS = 16
NW = NC * NS


def _argmin_body(x_ref, et_ref, idx_ref):
    x = x_ref[...]
    et = et_ref[...]
    mm = lax.dot_general(x, et, (((1,), (0,)), ((), ())),
                         preferred_element_type=jnp.float32)
    x2 = jnp.sum(x * x, axis=1, keepdims=True)
    e2 = jnp.sum(et * et, axis=0, keepdims=True)
    d2 = jnp.maximum(x2 + e2 - 2.0 * mm, 0.0)
    mn = jnp.min(d2, axis=1, keepdims=True)
    iota = lax.broadcasted_iota(jnp.int32, (TB, K), 1)
    cand = jnp.where(d2 == mn, iota, jnp.int32(K))
    idx = jnp.min(cand, axis=1)
    idx_ref[...] = idx.reshape(1, 1, TB)


def _tc_argmin(x, et):
    n = x.shape[0]
    nblk = n // TB
    return pl.pallas_call(
        _argmin_body,
        grid=(nblk,),
        in_specs=[
            pl.BlockSpec((TB, D), lambda i: (i, 0)),
            pl.BlockSpec((D, K), lambda i: (0, 0)),
        ],
        out_specs=pl.BlockSpec((1, 1, TB), lambda i: (i, 0, 0)),
        out_shape=jax.ShapeDtypeStruct((nblk, 1, TB), jnp.int32),
    )(x, et)


def _make_sc_stage(n_tokens):
    bpw = n_tokens // NW
    mesh = plsc.VectorSubcoreMesh(core_axis_name="c", subcore_axis_name="s")

    @functools.partial(
        pl.kernel,
        mesh=mesh,
        compiler_params=pltpu.CompilerParams(needs_layout_passes=False,
                                             use_tc_tiling_on_sc=False),
        out_type=[
            jax.ShapeDtypeStruct((n_tokens, D), jnp.float32),
            jax.ShapeDtypeStruct((n_tokens, D), jnp.float32),
            jax.ShapeDtypeStruct((NW, K), jnp.float32),
            jax.ShapeDtypeStruct((NW, 16), jnp.float32),
        ],
        scratch_types=[
            pltpu.VMEM((bpw,), jnp.int32),
            pltpu.VMEM((bpw, D), jnp.float32),
            pltpu.VMEM((bpw, D), jnp.float32),
            pltpu.VMEM((bpw, D), jnp.float32),
            pltpu.VMEM((K,), jnp.float32),
            pltpu.VMEM((16,), jnp.float32),
            pltpu.SemaphoreType.DMA,
        ],
    )
    def sc_stage(x_hbm, qacc_hbm, emb_hbm, idx_hbm,
                 xn_hbm, qn_hbm, fl_hbm, ssq_hbm,
                 idx_v, rows_v, x_v, qacc_v, flag_v, ssq_v, sem):
        wid = lax.axis_index("s") * NC + lax.axis_index("c")
        base = wid * bpw
        pltpu.sync_copy(idx_hbm.at[pl.ds(base, bpw)], idx_v)
        gather = pltpu.async_copy(emb_hbm.at[idx_v], rows_v, sem)
        pltpu.sync_copy(x_hbm.at[pl.ds(base, bpw)], x_v)
        pltpu.sync_copy(qacc_hbm.at[pl.ds(base, bpw)], qacc_v)
        gather.wait()

        def body(j, acc):
            q0 = rows_v[j, pl.ds(0, 16)]
            q1 = rows_v[j, pl.ds(16, 16)]
            x0 = x_v[j, pl.ds(0, 16)]
            x1 = x_v[j, pl.ds(16, 16)]
            r0 = x0 - q0
            r1 = x1 - q1
            x_v[j, pl.ds(0, 16)] = r0
            x_v[j, pl.ds(16, 16)] = r1
            qacc_v[j, pl.ds(0, 16)] = qacc_v[j, pl.ds(0, 16)] + q0
            qacc_v[j, pl.ds(16, 16)] = qacc_v[j, pl.ds(16, 16)] + q1
            return acc + r0 * r0 + r1 * r1

        ssq = lax.fori_loop(0, bpw, body, jnp.zeros((16,), jnp.float32))
        ssq_v[...] = ssq

        zeros16 = jnp.zeros((16,), jnp.float32)
        def zbody(i, carry):
            flag_v[pl.ds(i * 16, 16)] = zeros16
            return carry
        lax.fori_loop(0, K // 16, zbody, 0)

        ones16 = jnp.ones((16,), jnp.float32)
        def sbody(i, carry):
            iv = idx_v[pl.ds(i * 16, 16)]
            plsc.store_scatter(flag_v, [iv], ones16)
            return carry
        lax.fori_loop(0, bpw // 16, sbody, 0)

        pltpu.sync_copy(x_v, xn_hbm.at[pl.ds(base, bpw)])
        pltpu.sync_copy(qacc_v, qn_hbm.at[pl.ds(base, bpw)])
        pltpu.sync_copy(flag_v, fl_hbm.at[wid])
        pltpu.sync_copy(ssq_v, ssq_hbm.at[wid])

    return sc_stage


def _final_body(fl_ref, ssq_ref, loss_ref, util_ref, n_total):
    used = jnp.float32(0.0)
    for i in range(NCB):
        tot = jnp.sum(fl_ref[i], axis=0)
        used += jnp.sum((tot > 0.0).astype(jnp.float32))
    util_ref[0, 0] = used / (K * NCB)
    loss_ref[0, 0] = jnp.sum(ssq_ref[...]) * 2.0 / n_total / NCB


def _tc_final(flags, ssqs, n_total):
    body = functools.partial(_final_body, n_total=float(n_total))
    return pl.pallas_call(
        body,
        in_specs=[
            pl.BlockSpec((NCB, NW, K), lambda: (0, 0, 0)),
            pl.BlockSpec((16, 128), lambda: (0, 0)),
        ],
        out_specs=[
            pl.BlockSpec(memory_space=pltpu.SMEM),
            pl.BlockSpec(memory_space=pltpu.SMEM),
        ],
        out_shape=[
            jax.ShapeDtypeStruct((1, 1), jnp.float32),
            jax.ShapeDtypeStruct((1, 1), jnp.float32),
        ],
    )(flags, ssqs)


def kernel(z, embeds):
    bz, d, tz = z.shape
    n = bz * tz
    x = z.transpose(0, 2, 1).reshape(n, d)
    embeds_t = embeds.transpose(0, 2, 1)
    qacc = jnp.zeros_like(x)
    sc_stage = _make_sc_stage(n)

    idxs, flags, ssqs = [], [], []
    for i in range(NCB):
        idx = _tc_argmin(x, embeds_t[i]).reshape(n)
        x, qacc, fl, ssq = sc_stage(x, qacc, embeds[i], idx)
        idxs.append(idx)
        flags.append(fl)
        ssqs.append(ssq)

    loss, util = _tc_final(jnp.stack(flags),
                           jnp.stack(ssqs).reshape(16, 128),
                           n * d)
    quantized_total = qacc.reshape(bz, tz, d).transpose(0, 2, 1)
    all_indices = jnp.stack(idxs).reshape(NCB, bz, tz)
    return (quantized_total, all_indices, loss.reshape(()), util.reshape(()))


# speedup vs baseline: 2.3549x; 1.3646x over previous
import functools

import jax
import jax.numpy as jnp
from jax import lax
from jax.experimental import pallas as pl
from jax.experimental.pallas import tpu as pltpu
from jax.experimental.pallas import tpu_sc as plsc

D = 32
K = 8192
NCB = 4
TB = 256

NC = 2
NS = 16
NW = NC * NS


def _argmin_body(x_ref, et2_ref, idx_ref):
    x = x_ref[...]
    et2 = et2_ref[...]
    mm2 = lax.dot_general(x, et2, (((1,), (0,)), ((), ())),
                          preferred_element_type=jnp.float32)
    x2 = jnp.sum(x * x, axis=1, keepdims=True)
    e2 = 0.25 * jnp.sum(et2 * et2, axis=0, keepdims=True)
    d2 = x2 + e2 - mm2
    idx = jnp.argmin(d2, axis=1).astype(jnp.int32)
    idx_ref[...] = idx.reshape(1, 1, TB)


def _tc_argmin(x, et):
    n = x.shape[0]
    nblk = n // TB
    return pl.pallas_call(
        _argmin_body,
        grid=(nblk,),
        in_specs=[
            pl.BlockSpec((TB, D), lambda i: (i, 0)),
            pl.BlockSpec((D, K), lambda i: (0, 0)),
        ],
        out_specs=pl.BlockSpec((1, 1, TB), lambda i: (i, 0, 0)),
        out_shape=jax.ShapeDtypeStruct((nblk, 1, TB), jnp.int32),
    )(x, et)


def _make_sc_stage(n_tokens):
    bpw = n_tokens // NW
    mesh = plsc.VectorSubcoreMesh(core_axis_name="c", subcore_axis_name="s")

    @functools.partial(
        pl.kernel,
        mesh=mesh,
        compiler_params=pltpu.CompilerParams(needs_layout_passes=False,
                                             use_tc_tiling_on_sc=False),
        out_type=[
            jax.ShapeDtypeStruct((n_tokens, D), jnp.float32),
            jax.ShapeDtypeStruct((n_tokens, D), jnp.float32),
            jax.ShapeDtypeStruct((NW, K), jnp.float32),
            jax.ShapeDtypeStruct((NW, 16), jnp.float32),
        ],
        scratch_types=[
            pltpu.VMEM((bpw,), jnp.int32),
            pltpu.VMEM((bpw, D), jnp.float32),
            pltpu.VMEM((bpw, D), jnp.float32),
            pltpu.VMEM((bpw, D), jnp.float32),
            pltpu.VMEM((K,), jnp.float32),
            pltpu.VMEM((16,), jnp.float32),
            pltpu.SemaphoreType.DMA,
        ],
    )
    def sc_stage(x_hbm, qacc_hbm, emb_hbm, idx_hbm,
                 xn_hbm, qn_hbm, fl_hbm, ssq_hbm,
                 idx_v, rows_v, x_v, qacc_v, flag_v, ssq_v, sem):
        wid = lax.axis_index("s") * NC + lax.axis_index("c")
        base = wid * bpw
        pltpu.sync_copy(idx_hbm.at[pl.ds(base, bpw)], idx_v)
        gather = pltpu.async_copy(emb_hbm.at[idx_v], rows_v, sem)
        pltpu.sync_copy(x_hbm.at[pl.ds(base, bpw)], x_v)
        pltpu.sync_copy(qacc_hbm.at[pl.ds(base, bpw)], qacc_v)
        gather.wait()

        def body(j, acc):
            q0 = rows_v[j, pl.ds(0, 16)]
            q1 = rows_v[j, pl.ds(16, 16)]
            x0 = x_v[j, pl.ds(0, 16)]
            x1 = x_v[j, pl.ds(16, 16)]
            r0 = x0 - q0
            r1 = x1 - q1
            x_v[j, pl.ds(0, 16)] = r0
            x_v[j, pl.ds(16, 16)] = r1
            qacc_v[j, pl.ds(0, 16)] = qacc_v[j, pl.ds(0, 16)] + q0
            qacc_v[j, pl.ds(16, 16)] = qacc_v[j, pl.ds(16, 16)] + q1
            return acc + r0 * r0 + r1 * r1

        ssq = lax.fori_loop(0, bpw, body, jnp.zeros((16,), jnp.float32))
        ssq_v[...] = ssq

        zeros16 = jnp.zeros((16,), jnp.float32)
        def zbody(i, carry):
            flag_v[pl.ds(i * 16, 16)] = zeros16
            return carry
        lax.fori_loop(0, K // 16, zbody, 0)

        ones16 = jnp.ones((16,), jnp.float32)
        def sbody(i, carry):
            iv = idx_v[pl.ds(i * 16, 16)]
            plsc.store_scatter(flag_v, [iv], ones16)
            return carry
        lax.fori_loop(0, bpw // 16, sbody, 0)

        pltpu.sync_copy(x_v, xn_hbm.at[pl.ds(base, bpw)])
        pltpu.sync_copy(qacc_v, qn_hbm.at[pl.ds(base, bpw)])
        pltpu.sync_copy(flag_v, fl_hbm.at[wid])
        pltpu.sync_copy(ssq_v, ssq_hbm.at[wid])

    return sc_stage


def _final_body(fl_ref, ssq_ref, loss_ref, util_ref, n_total):
    used = jnp.float32(0.0)
    for i in range(NCB):
        tot = jnp.sum(fl_ref[i], axis=0)
        used += jnp.sum((tot > 0.0).astype(jnp.float32))
    util_ref[0, 0] = used / (K * NCB)
    loss_ref[0, 0] = jnp.sum(ssq_ref[...]) * 2.0 / n_total / NCB


def _tc_final(flags, ssqs, n_total):
    body = functools.partial(_final_body, n_total=float(n_total))
    return pl.pallas_call(
        body,
        in_specs=[
            pl.BlockSpec((NCB, NW, K), lambda: (0, 0, 0)),
            pl.BlockSpec((16, 128), lambda: (0, 0)),
        ],
        out_specs=[
            pl.BlockSpec(memory_space=pltpu.SMEM),
            pl.BlockSpec(memory_space=pltpu.SMEM),
        ],
        out_shape=[
            jax.ShapeDtypeStruct((1, 1), jnp.float32),
            jax.ShapeDtypeStruct((1, 1), jnp.float32),
        ],
    )(flags, ssqs)


def kernel(z, embeds):
    bz, d, tz = z.shape
    n = bz * tz
    x = z.transpose(0, 2, 1).reshape(n, d)
    embeds_t = embeds.transpose(0, 2, 1) * 2.0
    qacc = jnp.zeros_like(x)
    sc_stage = _make_sc_stage(n)

    idxs, flags, ssqs = [], [], []
    for i in range(NCB):
        idx = _tc_argmin(x, embeds_t[i]).reshape(n)
        x, qacc, fl, ssq = sc_stage(x, qacc, embeds[i], idx)
        idxs.append(idx)
        flags.append(fl)
        ssqs.append(ssq)

    loss, util = _tc_final(jnp.stack(flags),
                           jnp.stack(ssqs).reshape(16, 128),
                           n * d)
    quantized_total = qacc.reshape(bz, tz, d).transpose(0, 2, 1)
    all_indices = jnp.stack(idxs).reshape(NCB, bz, tz)
    return (quantized_total, all_indices, loss.reshape(()), util.reshape(()))
